# baseline (speedup 1.0000x reference)
import functools

import jax
import jax.numpy as jnp
from jax import lax
from jax.experimental import pallas as pl
from jax.experimental.pallas import tpu as pltpu
from jax.experimental.pallas import tpu_sc as plsc

SIDE = 256
SCALE = 3.0
NPTS = 1048576

NC = 2
NS = 16
NW = NC * NS
LANES = 16

CP = 128
PTS_PER_TILE = NPTS // NW
N_CHUNKS = PTS_PER_TILE // CP


def _gather_rows(table, idx_ref, dst_ref, sem):
    return pltpu.async_copy(table.at[idx_ref], dst_ref, sem)


def _sc_body(xs, ys, zs, table, biasv, out,
             xv, yv, zv, idx_s, w_s, par_s, dst, out_s, bias_v, sem):
    wid = lax.axis_index("s") * NC + lax.axis_index("c")
    tile_base = wid * PTS_PER_TILE

    pltpu.sync_copy(biasv, bias_v)
    iota = lax.iota(jnp.int32, LANES)
    colsel2 = lax.rem(iota, 4) * 2
    rowexp0 = lax.div(iota, 4)
    bvec = bias_v[...]

    def chunk(i, carry):
        base = tile_base + i * CP
        pltpu.sync_copy(xs.at[pl.ds(base, CP)], xv)
        pltpu.sync_copy(ys.at[pl.ds(base, CP)], yv)
        pltpu.sync_copy(zs.at[pl.ds(base, CP)], zv)

        for g in range(CP // LANES):
            sl = pl.ds(g * LANES, LANES)
            x = xv[sl]
            y = yv[sl]
            z = zv[sl]

            def axis(c):
                r = c / SCALE
                i_f = ((r + 1.0) * SIDE - 1.0) * 0.5
                i_f = jnp.minimum(jnp.maximum(i_f, 0.0), float(SIDE - 1))
                i0 = i_f.astype(jnp.int32)
                t = i_f - i0.astype(jnp.float32)
                return i0, t

            x0, tx = axis(x)
            y0, ty = axis(y)
            z0, tz = axis(z)
            dx = jnp.where(x0 < SIDE - 1, 1, 0)
            x1 = x0 + dx
            xh0 = lax.shift_right_logical(x0, 1)
            xh1 = lax.shift_right_logical(x1, 1)
            par0 = lax.bitwise_and(x0, 1)
            par1 = lax.bitwise_and(x1, 1)
            dy2 = jnp.where(y0 < SIDE - 1, SIDE // 2, 0)
            dz2 = jnp.where(z0 < SIDE - 1, (SIDE * SIDE) // 2, 0)
            zyb = (z0 * SIDE + y0) * (SIDE // 2)
            wx0 = 1.0 - tx
            wy0 = 1.0 - ty
            wz0 = 1.0 - tz

            zybs = (zyb, zyb + dy2, zyb + dz2, zyb + dz2 + dy2)
            azys = (wz0 * wy0, wz0 * ty, tz * wy0, tz * ty)
            off = g * LANES
            par_s[pl.ds(off, LANES)] = par0
            par_s[pl.ds(CP + off, LANES)] = par1
            for zy in range(4):
                for side in range(2):
                    gi = zy * 2 + side
                    idx_s[pl.ds(gi * CP + off, LANES)] = (
                        zybs[zy] + (xh1 if side else xh0))
                    w_s[pl.ds(gi * CP + off, LANES)] = (
                        azys[zy] * (tx if side else wx0))

        copies = []
        for gi in range(8):
            copies.append(
                _gather_rows(
                    table,
                    idx_s.at[pl.ds(gi * CP, CP)],
                    dst.at[pl.ds(gi * CP, CP)],
                    sem,
                )
            )
        for cp in copies:
            cp.wait()

        for j in range(CP // 4):
            rowe = rowexp0 + (4 * j)
            col0 = colsel2 + plsc.load_gather(par_s, [rowe])
            col1 = colsel2 + plsc.load_gather(par_s, [rowe + CP])
            acc = bvec
            for zy in range(4):
                for side in range(2):
                    gi = zy * 2 + side
                    r = rowe + (gi * CP)
                    w = plsc.load_gather(w_s, [r])
                    v = plsc.load_gather(dst, [r, col1 if side else col0])
                    acc = acc + w * v
            out_s[pl.ds(j * LANES, LANES)] = acc

        pltpu.sync_copy(out_s, out.at[pl.ds(base * 4, CP * 4)])
        return carry

    lax.fori_loop(0, N_CHUNKS, chunk, 0, unroll=False)


@jax.jit
def _voxel_sample(xs, ys, zs, table, biasv):
    mesh = plsc.VectorSubcoreMesh(
        core_axis_name="c", subcore_axis_name="s",
        num_cores=NC, num_subcores=NS)
    f = pl.kernel(
        functools.partial(_sc_body),
        out_type=jax.ShapeDtypeStruct((NPTS * 4,), jnp.float32),
        mesh=mesh,
        scratch_types=[
            pltpu.VMEM((CP,), jnp.float32),
            pltpu.VMEM((CP,), jnp.float32),
            pltpu.VMEM((CP,), jnp.float32),
            pltpu.VMEM((8 * CP,), jnp.int32),
            pltpu.VMEM((8 * CP,), jnp.float32),
            pltpu.VMEM((2 * CP,), jnp.int32),
            pltpu.VMEM((8 * CP, 8), jnp.float32),
            pltpu.VMEM((CP * 4,), jnp.float32),
            pltpu.VMEM((LANES,), jnp.float32),
            pltpu.SemaphoreType.DMA,
        ],
        compiler_params=pltpu.CompilerParams(
            needs_layout_passes=False, use_tc_tiling_on_sc=False),
    )
    return f(xs, ys, zs, table, biasv)


def kernel(positions, voxels, bias):
    vp = voxels[0].reshape(4, (SIDE * SIDE * SIDE) // 2, 2)
    table = jnp.stack((vp[0], vp[1], vp[2], vp[3]), axis=1).reshape(-1, 8)
    xs = positions[:, 0]
    ys = positions[:, 1]
    zs = positions[:, 2]
    biasv = jnp.tile(bias[0], 4)
    flat = _voxel_sample(xs, ys, zs, table, biasv)
    return flat.reshape(NPTS, 4)


# speedup vs baseline: 30.3544x; 30.3544x over previous
import functools

import jax
import jax.numpy as jnp
from jax import lax
from jax.experimental import pallas as pl
from jax.experimental.pallas import tpu as pltpu
from jax.experimental.pallas import tpu_sc as plsc

SIDE = 256
SCALE = 3.0
NPTS = 1048576
NVOX = SIDE * SIDE * SIDE

NC = 2
NS = 16
NW = NC * NS
LANES = 16

CP = 128
PTS_PER_TILE = NPTS // NW
N_CHUNKS = PTS_PER_TILE // CP

ZY = SIDE * SIDE
ZY_PER_TILE = ZY // NW
BROWS = 16
ROWF = 4 * SIDE


def _build_body(src, table, in_v, out_v, pat_s, sem):
    wid = lax.axis_index("s") * NC + lax.axis_index("c")
    row_base = wid * ZY_PER_TILE

    iota = lax.iota(jnp.int32, LANES)
    cpat = lax.bitwise_and(lax.shift_right_logical(iota, 1), 3)
    xpat = (lax.shift_left(lax.shift_right_logical(iota, 3), 1)
            + lax.bitwise_and(iota, 1))
    pat_s[pl.ds(0, LANES)] = cpat
    pat_s[pl.ds(LANES, LANES)] = xpat

    def chunk(i, carry):
        r0 = row_base + i * BROWS
        z = r0 // SIDE
        y0 = r0 % SIDE
        pltpu.sync_copy(src.at[0, :, z, pl.ds(y0, BROWS), :], in_v)
        cp = pat_s[pl.ds(0, LANES)]
        xp = pat_s[pl.ds(LANES, LANES)]
        for b in range(BROWS):
            yi = jnp.full((LANES,), b, jnp.int32)
            for j in range(ROWF // LANES):
                out_v[pl.ds(b * ROWF + j * LANES, LANES)] = (
                    plsc.load_gather(in_v, [cp, yi, xp + 4 * j]))
        pltpu.sync_copy(out_v, table.at[pl.ds(r0 * ROWF, BROWS * ROWF)])
        return carry

    lax.fori_loop(0, ZY_PER_TILE // BROWS, chunk, 0, unroll=False)


def _build_table(voxels):
    mesh = plsc.VectorSubcoreMesh(
        core_axis_name="c", subcore_axis_name="s",
        num_cores=NC, num_subcores=NS)
    f = pl.kernel(
        _build_body,
        out_type=jax.ShapeDtypeStruct((NVOX * 4,), jnp.float32),
        mesh=mesh,
        scratch_types=[
            pltpu.VMEM((4, BROWS, SIDE), jnp.float32),
            pltpu.VMEM((BROWS * ROWF,), jnp.float32),
            pltpu.VMEM((2 * LANES,), jnp.int32),
            pltpu.SemaphoreType.DMA,
        ],
        compiler_params=pltpu.CompilerParams(
            needs_layout_passes=False, use_tc_tiling_on_sc=False),
    )
    return f(voxels)


def _gather_rows(table, idx_ref, dst_ref, sem):
    return pltpu.async_copy(table.at[idx_ref], dst_ref, sem)


def _sc_body(xs, ys, zs, table, biasv, out,
             xv, yv, zv, idx_s, w_s, par_s, dst, out_s, bias_v, sem):
    wid = lax.axis_index("s") * NC + lax.axis_index("c")
    tile_base = wid * PTS_PER_TILE

    pltpu.sync_copy(biasv, bias_v)
    iota = lax.iota(jnp.int32, LANES)
    colsel2 = lax.rem(iota, 4) * 2
    rowexp0 = lax.div(iota, 4)
    bvec = bias_v[...]

    def chunk(i, carry):
        base = tile_base + i * CP
        pltpu.sync_copy(xs.at[pl.ds(base, CP)], xv)
        pltpu.sync_copy(ys.at[pl.ds(base, CP)], yv)
        pltpu.sync_copy(zs.at[pl.ds(base, CP)], zv)

        for g in range(CP // LANES):
            sl = pl.ds(g * LANES, LANES)
            x = xv[sl]
            y = yv[sl]
            z = zv[sl]

            def axis(c):
                r = c / SCALE
                i_f = ((r + 1.0) * SIDE - 1.0) * 0.5
                i_f = jnp.minimum(jnp.maximum(i_f, 0.0), float(SIDE - 1))
                i0 = i_f.astype(jnp.int32)
                t = i_f - i0.astype(jnp.float32)
                return i0, t

            x0, tx = axis(x)
            y0, ty = axis(y)
            z0, tz = axis(z)
            dx = jnp.where(x0 < SIDE - 1, 1, 0)
            x1 = x0 + dx
            xh0 = lax.shift_right_logical(x0, 1)
            xh1 = lax.shift_right_logical(x1, 1)
            par0 = lax.bitwise_and(x0, 1)
            par1 = lax.bitwise_and(x1, 1)
            dy2 = jnp.where(y0 < SIDE - 1, SIDE // 2, 0)
            dz2 = jnp.where(z0 < SIDE - 1, (SIDE * SIDE) // 2, 0)
            zyb = (z0 * SIDE + y0) * (SIDE // 2)
            wx0 = 1.0 - tx
            wy0 = 1.0 - ty
            wz0 = 1.0 - tz

            zybs = (zyb, zyb + dy2, zyb + dz2, zyb + dz2 + dy2)
            azys = (wz0 * wy0, wz0 * ty, tz * wy0, tz * ty)
            off = g * LANES
            par_s[pl.ds(off, LANES)] = par0
            par_s[pl.ds(CP + off, LANES)] = par1
            for zy in range(4):
                for side in range(2):
                    gi = zy * 2 + side
                    idx_s[pl.ds(gi * CP + off, LANES)] = (
                        zybs[zy] + (xh1 if side else xh0))
                    w_s[pl.ds(gi * CP + off, LANES)] = (
                        azys[zy] * (tx if side else wx0))

        copies = []
        for gi in range(8):
            copies.append(
                _gather_rows(
                    table,
                    idx_s.at[pl.ds(gi * CP, CP)],
                    dst.at[pl.ds(gi * CP, CP)],
                    sem,
                )
            )
        for cp in copies:
            cp.wait()

        for j in range(CP // 4):
            rowe = rowexp0 + (4 * j)
            col0 = colsel2 + plsc.load_gather(par_s, [rowe])
            col1 = colsel2 + plsc.load_gather(par_s, [rowe + CP])
            acc = bvec
            for zy in range(4):
                for side in range(2):
                    gi = zy * 2 + side
                    r = rowe + (gi * CP)
                    w = plsc.load_gather(w_s, [r])
                    v = plsc.load_gather(dst, [r, col1 if side else col0])
                    acc = acc + w * v
            out_s[pl.ds(j * LANES, LANES)] = acc

        pltpu.sync_copy(out_s, out.at[pl.ds(base * 4, CP * 4)])
        return carry

    lax.fori_loop(0, N_CHUNKS, chunk, 0, unroll=False)


@jax.jit
def _voxel_sample(xs, ys, zs, table, biasv):
    mesh = plsc.VectorSubcoreMesh(
        core_axis_name="c", subcore_axis_name="s",
        num_cores=NC, num_subcores=NS)
    f = pl.kernel(
        functools.partial(_sc_body),
        out_type=jax.ShapeDtypeStruct((NPTS * 4,), jnp.float32),
        mesh=mesh,
        scratch_types=[
            pltpu.VMEM((CP,), jnp.float32),
            pltpu.VMEM((CP,), jnp.float32),
            pltpu.VMEM((CP,), jnp.float32),
            pltpu.VMEM((8 * CP,), jnp.int32),
            pltpu.VMEM((8 * CP,), jnp.float32),
            pltpu.VMEM((2 * CP,), jnp.int32),
            pltpu.VMEM((8 * CP, 8), jnp.float32),
            pltpu.VMEM((CP * 4,), jnp.float32),
            pltpu.VMEM((LANES,), jnp.float32),
            pltpu.SemaphoreType.DMA,
        ],
        compiler_params=pltpu.CompilerParams(
            needs_layout_passes=False, use_tc_tiling_on_sc=False),
    )
    return f(xs, ys, zs, table, biasv)


def kernel(positions, voxels, bias):
    table = _build_table(voxels).reshape(NVOX // 2, 8)
    xs = positions[:, 0]
    ys = positions[:, 1]
    zs = positions[:, 2]
    biasv = jnp.tile(bias[0], 4)
    flat_out = _voxel_sample(xs, ys, zs, table, biasv)
    return flat_out.reshape(NPTS, 4)
